# Initial kernel scaffold; baseline (speedup 1.0000x reference)
import jax
import jax.numpy as jnp
from jax.experimental import pallas as pl


def kernel(feature_maps, intrinstics, W_depthnet, b_depthnet):
    raise NotImplementedError("write your pallas kernel here")


# speedup vs baseline: 43.2204x; 43.2204x over previous
import functools

import jax
import jax.numpy as jnp
from jax import lax
from jax.experimental import pallas as pl
from jax.experimental.pallas import tpu as pltpu
from jax.experimental.pallas import tpu_sc as plsc

B = 4
IN_C = 256
FH = 64
FW = 64
HW = FH * FW
D = 49
DP = 64
C = 64
NX0 = 100
NX2 = 98
NVOX = NX2 * NX0
SLAB = 20480
DUMP = 2 * NVOX
NPIX = B * HW
PIX_PER_TILE = 512
ROWS_PER_TILE = SLAB // 16


def _tc_body(inv_ref, fm_ref, w_ref, b_ref, xy_ref, fc_ref, wo_ref, vid_ref):
    feat = jnp.dot(w_ref[...], fm_ref[0], preferred_element_type=jnp.float32)
    feat = feat + b_ref[...]
    fc_ref[0] = feat[64:128]

    f56 = feat[0:56]
    drow = lax.broadcasted_iota(jnp.int32, (56, 1), 0)
    f56 = jnp.where(drow < D, f56, -1e30)
    m = jnp.max(f56, axis=0, keepdims=True)
    e = jnp.exp(f56 - m)
    s = jnp.sum(e, axis=0, keepdims=True)
    depth = e / s

    xs = xy_ref[0:1, :]
    ys = xy_ref[1:2, :]
    bmod = pl.program_id(0) % 2
    i00 = inv_ref[0, 0, 0]; i01 = inv_ref[0, 0, 1]; i02 = inv_ref[0, 0, 2]
    i10 = inv_ref[0, 0, 3]; i11 = inv_ref[0, 0, 4]; i12 = inv_ref[0, 0, 5]
    i20 = inv_ref[0, 0, 6]; i21 = inv_ref[0, 0, 7]; i22 = inv_ref[0, 0, 8]

    for blk in range(7):
        dv = (lax.broadcasted_iota(jnp.int32, (8, 1), 0)
              .astype(jnp.float32) + (1.0 + 8.0 * blk))
        px = (xs * dv).astype(jnp.bfloat16).astype(jnp.float32)
        py = (ys * dv).astype(jnp.bfloat16).astype(jnp.float32)
        pz = jnp.broadcast_to(dv, (8, HW))
        gxf = (i00 * px + i01 * py + i02 * pz + 25.0) / 0.5
        gyf = (i10 * px + i11 * py + i12 * pz + 10.0) / 20.0
        gzf = (i20 * px + i21 * py + i22 * pz - 1.0) / 0.5
        gx = gxf.astype(jnp.int32)
        gy = gyf.astype(jnp.int32)
        gz = gzf.astype(jnp.int32)
        kept = ((gx >= 0) & (gx < NX0) & (gy >= 0) & (gy < 1)
                & (gz >= 0) & (gz < NX2) & (dv <= 49.0))
        dep = depth[8 * blk:8 * blk + 8]
        wo_ref[0, 8 * blk:8 * blk + 8, :] = jnp.where(kept, dep, 0.0)
        vid_ref[0, 8 * blk:8 * blk + 8, :] = jnp.where(
            kept, gz * NX0 + gx + bmod * NVOX, DUMP)

    wo_ref[0, 56:64, :] = jnp.zeros((8, HW), jnp.float32)
    vid_ref[0, 56:64, :] = jnp.full((8, HW), DUMP, jnp.int32)


def _tc_stage(fm_r, inv9, w_pad, b_pad, xy):
    return pl.pallas_call(
        _tc_body,
        grid=(B,),
        in_specs=[
            pl.BlockSpec((1, 1, 16), lambda b: (b, 0, 0),
                         memory_space=pltpu.SMEM),
            pl.BlockSpec((1, IN_C, HW), lambda b: (b, 0, 0)),
            pl.BlockSpec((128, IN_C), lambda b: (0, 0)),
            pl.BlockSpec((128, 1), lambda b: (0, 0)),
            pl.BlockSpec((8, HW), lambda b: (0, 0)),
        ],
        out_specs=[
            pl.BlockSpec((1, C, HW), lambda b: (b, 0, 0)),
            pl.BlockSpec((1, DP, HW), lambda b: (b, 0, 0)),
            pl.BlockSpec((1, DP, HW), lambda b: (b, 0, 0)),
        ],
        out_shape=[
            jax.ShapeDtypeStruct((B, C, HW), jnp.float32),
            jax.ShapeDtypeStruct((B, DP, HW), jnp.float32),
            jax.ShapeDtypeStruct((B, DP, HW), jnp.int32),
        ],
        compiler_params=pltpu.CompilerParams(
            dimension_semantics=("arbitrary",)),
    )(inv9, fm_r, w_pad, b_pad, xy)


CH = 32


def _sc_body(f_hbm, w_hbm, vid_hbm, out_hbm,
             f_c, w_c, vid_c, sc_v, idx_v, slab):
    cid = lax.axis_index("c")
    sid = lax.axis_index("s")
    zbase = sid * ROWS_PER_TILE

    zvec = jnp.zeros((16,), jnp.float32)

    def zrow(i, _):
        for c4 in range(4):
            sc_v[i, pl.ds(c4 * 16, 16)] = zvec
        return 0
    lax.fori_loop(0, DP, zrow, 0)

    def zchunk(k, _):
        pltpu.sync_copy(sc_v, slab.at[pl.ds(zbase + k * DP, DP)])
        return 0
    lax.fori_loop(0, ROWS_PER_TILE // DP, zchunk, 0)
    plsc.subcore_barrier()

    g0 = (cid * 16 + sid) * PIX_PER_TILE

    def chunk_body(ck, _):
        p0 = g0 + ck * CH
        pltpu.sync_copy(f_hbm.at[pl.ds(p0, CH)], f_c)
        pltpu.sync_copy(w_hbm.at[pl.ds(p0, CH)], w_c)
        pltpu.sync_copy(vid_hbm.at[pl.ds(p0, CH)], vid_c)

        def pix_body(q, _):
            for c4 in range(4):
                idx_v[pl.ds(c4 * 16, 16)] = vid_c[q, pl.ds(c4 * 16, 16)]
            fv = [f_c[q, pl.ds(c4 * 16, 16)] for c4 in range(4)]
            for dg in range(4):
                wv = w_c[q, pl.ds(dg * 16, 16)]
                for j in range(16):
                    ws = wv[j]
                    for c4 in range(4):
                        sc_v[dg * 16 + j, pl.ds(c4 * 16, 16)] = ws * fv[c4]
            pltpu.sync_copy(sc_v, slab.at[idx_v], add=True)
            return 0
        lax.fori_loop(0, CH, pix_body, 0)
        return 0
    lax.fori_loop(0, PIX_PER_TILE // CH, chunk_body, 0)

    plsc.subcore_barrier()
    pltpu.sync_copy(slab.at[pl.ds(zbase, ROWS_PER_TILE)],
                    out_hbm.at[cid, pl.ds(zbase, ROWS_PER_TILE)])


_sc_stage = functools.partial(
    pl.kernel,
    out_type=jax.ShapeDtypeStruct((2, SLAB, C), jnp.float32),
    mesh=plsc.VectorSubcoreMesh(core_axis_name="c", subcore_axis_name="s"),
    scratch_types=[
        pltpu.VMEM((CH, C), jnp.float32),
        pltpu.VMEM((CH, DP), jnp.float32),
        pltpu.VMEM((CH, DP), jnp.int32),
        pltpu.VMEM((DP, C), jnp.float32),
        pltpu.VMEM((DP,), jnp.int32),
        pltpu.VMEM_SHARED((SLAB, C), jnp.float32),
    ],
    compiler_params=pltpu.CompilerParams(use_tc_tiling_on_sc=False),
)(_sc_body)


@jax.jit
def kernel(feature_maps, intrinstics, W_depthnet, b_depthnet):
    fm_r = feature_maps.reshape(B, IN_C, HW).astype(jnp.bfloat16)
    inv = jnp.linalg.inv(intrinstics).astype(jnp.bfloat16).astype(jnp.float32)
    inv9 = jnp.concatenate(
        [inv.reshape(B, 9), jnp.zeros((B, 7), jnp.float32)],
        axis=1).reshape(B, 1, 16)

    w_pad = jnp.zeros((128, IN_C), jnp.float32)
    w_pad = w_pad.at[0:D].set(W_depthnet[0:D])
    w_pad = w_pad.at[64:64 + C].set(W_depthnet[D:D + C])
    w_pad = w_pad.astype(jnp.bfloat16)
    b_pad = jnp.zeros((128, 1), jnp.float32)
    b_pad = b_pad.at[0:D, 0].set(b_depthnet[0:D])
    b_pad = b_pad.at[64:64 + C, 0].set(b_depthnet[D:D + C])

    xs = jnp.linspace(0.0, 255.0, FW, dtype=jnp.float32)
    ys = jnp.linspace(0.0, 255.0, FH, dtype=jnp.float32)
    xg = jnp.broadcast_to(xs[None, :], (FH, FW)).reshape(HW)
    yg = jnp.broadcast_to(ys[:, None], (FH, FW)).reshape(HW)
    xy = jnp.zeros((8, HW), jnp.float32).at[0].set(xg).at[1].set(yg)

    featc, wout, vidout = _tc_stage(fm_r, inv9, w_pad, b_pad, xy)

    f_pix = featc.transpose(0, 2, 1).reshape(NPIX, C)
    w_pix = wout.transpose(0, 2, 1).reshape(NPIX, DP)
    vid_pix = vidout.transpose(0, 2, 1).reshape(NPIX, DP)

    slabs = _sc_stage(f_pix, w_pix, vid_pix)

    bev = slabs[:, :2 * NVOX].reshape(B, NVOX, C)
    return bev.reshape(B, NX2, NX0, C).transpose(0, 3, 1, 2)
